# baseline (speedup 1.0000x reference)
import functools

import numpy as np
import jax
import jax.numpy as jnp
from jax.experimental import pallas as pl
from jax.experimental.pallas import tpu as pltpu

_KSIZE = 4
_SLOPE = 0.01


def _flat_kernel(x_ref, t1_ref, b1_ref, t2_ref, b2_ref, t3_ref, b3_ref,
                 wl_ref, bl_ref, o_ref, *, C, H, W, oh1, oh2, oh3):
    f32 = jnp.float32
    bf16 = jnp.bfloat16

    def lrelu(v):
        return jnp.where(v > 0, v, _SLOPE * v)

    n1 = t1_ref.shape[1]
    n2 = t2_ref.shape[1]
    n3 = t3_ref.shape[1]

    u = x_ref[0, 0].astype(bf16)

    a1_parts = []
    for r in range(oh1):
        acc = b1_ref[...].astype(f32)
        for c in range(C):
            lo = c * H * W + 2 * r * W
            acc = acc + jnp.dot(u[:, lo:lo + _KSIZE * W],
                                t1_ref[c * _KSIZE * W:(c + 1) * _KSIZE * W, :],
                                preferred_element_type=f32)
        a1_parts.append(lrelu(acc).astype(bf16))
    a1 = jnp.concatenate(a1_parts, axis=1)

    a2_parts = [
        lrelu(jnp.dot(a1[:, r * n1:(r + _KSIZE) * n1], t2_ref[...],
                      preferred_element_type=f32) + b2_ref[...]).astype(bf16)
        for r in range(oh2)]
    a2 = jnp.concatenate(a2_parts, axis=1)

    a3_parts = [
        lrelu(jnp.dot(a2[:, r * n2:(r + _KSIZE) * n2], t3_ref[...],
                      preferred_element_type=f32) + b3_ref[...]).astype(bf16)
        for r in range(oh3)]
    a3 = jnp.concatenate(a3_parts, axis=1)

    y = jnp.dot(a3, wl_ref[...], preferred_element_type=f32) + bl_ref[...]
    o_ref[0, 0] = y


def kernel(x, t1, b1, t2, b2, t3, b3, wl, bl):
    B, C, H, W = x.shape
    oh1 = (H - _KSIZE) // 2 + 1
    oh2 = oh1 - (_KSIZE - 1)
    oh3 = oh2 - (_KSIZE - 1)
    wcp = t1.shape[0] // _KSIZE
    no = bl.shape[1]

    ncores = 1
    block_b = max(1, min(256, -(-B // 2)))
    block_b = min(block_b, B)
    grid_b = -(-B // block_b)
    bp = grid_b * block_b
    grid_i = grid_b

    perm = np.zeros(C * _KSIZE * W, np.int32)
    for c in range(C):
        for kh in range(_KSIZE):
            for w in range(W):
                perm[c * _KSIZE * W + kh * W + w] = kh * wcp + w * C + c
    t1p = t1[jnp.asarray(perm), :]

    xf = x.reshape(B, C * H * W)
    if bp > B:
        xf = jnp.pad(xf, ((0, bp - B), (0, 0)))
    xs = xf.reshape(ncores, grid_i, block_b, C * H * W)

    body = functools.partial(_flat_kernel, C=C, H=H, W=W,
                             oh1=oh1, oh2=oh2, oh3=oh3)

    ow1 = (W - _KSIZE) // 2 + 1
    ow2 = ow1 - (_KSIZE - 1)
    ow3 = ow2 - (_KSIZE - 1)
    flops = 2 * B * (oh1 * ow1 * 16 * (C * _KSIZE * _KSIZE)
                     + oh2 * ow2 * 32 * (16 * _KSIZE * _KSIZE)
                     + oh3 * ow3 * 64 * (32 * _KSIZE * _KSIZE)
                     + no * (64 * oh3 * ow3))
    bytes_accessed = (int(np.prod(xs.shape)) * 4 + bp * no * 4
                      + sum(int(a.size) * a.dtype.itemsize
                            for a in (t1, b1, t2, b2, t3, b3, wl, bl)))

    def full(a):
        nd = a.ndim
        return pl.BlockSpec(a.shape, lambda c, g, _nd=nd: (0,) * _nd)

    out = pl.pallas_call(
        body,
        out_shape=jax.ShapeDtypeStruct((ncores, grid_i, block_b, no),
                                       jnp.float32),
        grid=(ncores, grid_i),
        in_specs=[
            pl.BlockSpec((1, 1, block_b, C * H * W),
                         lambda c, g: (c, g, 0, 0)),
            full(t1p), full(b1), full(t2), full(b2), full(t3), full(b3),
            full(wl), full(bl),
        ],
        out_specs=pl.BlockSpec((1, 1, block_b, no),
                               lambda c, g: (c, g, 0, 0)),
        compiler_params=pltpu.CompilerParams(
            dimension_semantics=("arbitrary", "arbitrary")),
        cost_estimate=pl.CostEstimate(flops=flops, transcendentals=0,
                                      bytes_accessed=bytes_accessed),
    )(xs, t1p, b1, t2, b2, t3, b3, wl, bl)

    return out.reshape(bp, no)[:B, :no]


# speedup vs baseline: 1.0005x; 1.0005x over previous
import functools

import numpy as np
import jax
import jax.numpy as jnp
from jax.experimental import pallas as pl
from jax.experimental.pallas import tpu as pltpu

_KSIZE = 4
_SLOPE = 0.01


def _flat_kernel(x_ref, t1_ref, b1_ref, t2_ref, b2_ref, t3_ref, b3_ref,
                 wl_ref, bl_ref, o_ref, *, C, H, W, oh1, oh2, oh3):
    f32 = jnp.float32
    bf16 = jnp.bfloat16

    def lrelu(v):
        return jnp.where(v > 0, v, _SLOPE * v)

    n1 = t1_ref.shape[1]
    n2 = t2_ref.shape[1]
    n3 = t3_ref.shape[1]

    u = x_ref[0, 0].astype(bf16)

    a1_parts = []
    for r in range(oh1):
        acc = b1_ref[...].astype(f32)
        for c in range(C):
            lo = c * H * W + 2 * r * W
            acc = acc + jnp.dot(u[:, lo:lo + _KSIZE * W],
                                t1_ref[c * _KSIZE * W:(c + 1) * _KSIZE * W, :],
                                preferred_element_type=f32)
        a1_parts.append(lrelu(acc).astype(bf16))
    a1 = jnp.concatenate(a1_parts, axis=1)

    a2_parts = [
        lrelu(jnp.dot(a1[:, r * n1:(r + _KSIZE) * n1], t2_ref[...],
                      preferred_element_type=f32) + b2_ref[...]).astype(bf16)
        for r in range(oh2)]
    a2 = jnp.concatenate(a2_parts, axis=1)

    a3_parts = [
        lrelu(jnp.dot(a2[:, r * n2:(r + _KSIZE) * n2], t3_ref[...],
                      preferred_element_type=f32) + b3_ref[...]).astype(bf16)
        for r in range(oh3)]
    a3 = jnp.concatenate(a3_parts, axis=1)

    y = jnp.dot(a3, wl_ref[...], preferred_element_type=f32) + bl_ref[...]
    o_ref[0, 0] = y


def kernel(x, t1, b1, t2, b2, t3, b3, wl, bl):
    B, C, H, W = x.shape
    oh1 = (H - _KSIZE) // 2 + 1
    oh2 = oh1 - (_KSIZE - 1)
    oh3 = oh2 - (_KSIZE - 1)
    wcp = t1.shape[0] // _KSIZE
    no = bl.shape[1]

    ncores = 1
    block_b = max(1, min(256, -(-B // 2)))
    block_b = min(block_b, B)
    grid_b = -(-B // block_b)
    bp = grid_b * block_b
    grid_i = grid_b

    n1 = t1.shape[1]
    t1p = (t1.reshape(_KSIZE, wcp, n1)[:, :C * W, :]
           .reshape(_KSIZE, W, C, n1)
           .transpose(2, 0, 1, 3)
           .reshape(C * _KSIZE * W, n1))

    xf = x.reshape(B, C * H * W)
    if bp > B:
        xf = jnp.pad(xf, ((0, bp - B), (0, 0)))
    xs = xf.reshape(ncores, grid_i, block_b, C * H * W)

    body = functools.partial(_flat_kernel, C=C, H=H, W=W,
                             oh1=oh1, oh2=oh2, oh3=oh3)

    ow1 = (W - _KSIZE) // 2 + 1
    ow2 = ow1 - (_KSIZE - 1)
    ow3 = ow2 - (_KSIZE - 1)
    flops = 2 * B * (oh1 * ow1 * 16 * (C * _KSIZE * _KSIZE)
                     + oh2 * ow2 * 32 * (16 * _KSIZE * _KSIZE)
                     + oh3 * ow3 * 64 * (32 * _KSIZE * _KSIZE)
                     + no * (64 * oh3 * ow3))
    bytes_accessed = (int(np.prod(xs.shape)) * 4 + bp * no * 4
                      + sum(int(a.size) * a.dtype.itemsize
                            for a in (t1, b1, t2, b2, t3, b3, wl, bl)))

    def full(a):
        nd = a.ndim
        return pl.BlockSpec(a.shape, lambda c, g, _nd=nd: (0,) * _nd)

    out = pl.pallas_call(
        body,
        out_shape=jax.ShapeDtypeStruct((ncores, grid_i, block_b, no),
                                       jnp.float32),
        grid=(ncores, grid_i),
        in_specs=[
            pl.BlockSpec((1, 1, block_b, C * H * W),
                         lambda c, g: (c, g, 0, 0)),
            full(t1p), full(b1), full(t2), full(b2), full(t3), full(b3),
            full(wl), full(bl),
        ],
        out_specs=pl.BlockSpec((1, 1, block_b, no),
                               lambda c, g: (c, g, 0, 0)),
        compiler_params=pltpu.CompilerParams(
            dimension_semantics=("arbitrary", "arbitrary")),
        cost_estimate=pl.CostEstimate(flops=flops, transcendentals=0,
                                      bytes_accessed=bytes_accessed),
    )(xs, t1p, b1, t2, b2, t3, b3, wl, bl)

    return out.reshape(bp, no)[:B, :no]


# speedup vs baseline: 1.0109x; 1.0104x over previous
import functools

import numpy as np
import jax
import jax.numpy as jnp
from jax.experimental import pallas as pl
from jax.experimental.pallas import tpu as pltpu

_KSIZE = 4
_SLOPE = 0.01


def _flat_kernel(x_ref, t1_ref, b1_ref, t2_ref, b2_ref, t3_ref, b3_ref,
                 wl_ref, bl_ref, o_ref, *, C, H, W, oh1, oh2, oh3):
    f32 = jnp.float32
    bf16 = jnp.bfloat16

    def lrelu(v):
        return jnp.where(v > 0, v, _SLOPE * v)

    n1 = t1_ref.shape[1]
    n2 = t2_ref.shape[1]
    n3 = t3_ref.shape[1]

    u = x_ref[0, 0]

    a1_parts = []
    for r in range(oh1):
        acc = b1_ref[...].astype(f32)
        for c in range(C):
            lo = c * H * W + 2 * r * W
            acc = acc + jnp.dot(u[:, lo:lo + _KSIZE * W],
                                t1_ref[c * _KSIZE * W:(c + 1) * _KSIZE * W, :],
                                preferred_element_type=f32)
        a1_parts.append(lrelu(acc).astype(bf16))
    a1 = jnp.concatenate(a1_parts, axis=1)

    a2_parts = [
        lrelu(jnp.dot(a1[:, r * n1:(r + _KSIZE) * n1], t2_ref[...],
                      preferred_element_type=f32) + b2_ref[...]).astype(bf16)
        for r in range(oh2)]
    a2 = jnp.concatenate(a2_parts, axis=1)

    a3_parts = [
        lrelu(jnp.dot(a2[:, r * n2:(r + _KSIZE) * n2], t3_ref[...],
                      preferred_element_type=f32) + b3_ref[...]).astype(bf16)
        for r in range(oh3)]
    a3 = jnp.concatenate(a3_parts, axis=1)

    y = jnp.dot(a3, wl_ref[...], preferred_element_type=f32) + bl_ref[...]
    o_ref[0, 0] = y


def kernel(x, t1, b1, t2, b2, t3, b3, wl, bl):
    B, C, H, W = x.shape
    oh1 = (H - _KSIZE) // 2 + 1
    oh2 = oh1 - (_KSIZE - 1)
    oh3 = oh2 - (_KSIZE - 1)
    wcp = t1.shape[0] // _KSIZE
    no = bl.shape[1]

    ncores = 1
    block_b = max(1, min(256, -(-B // 2)))
    block_b = min(block_b, B)
    grid_b = -(-B // block_b)
    bp = grid_b * block_b
    grid_i = grid_b

    n1 = t1.shape[1]
    t1p = (t1.reshape(_KSIZE, wcp, n1)[:, :C * W, :]
           .reshape(_KSIZE, W, C, n1)
           .transpose(2, 0, 1, 3)
           .reshape(C * _KSIZE * W, n1))

    xf = x.reshape(B, C * H * W).astype(jnp.bfloat16)
    if bp > B:
        xf = jnp.pad(xf, ((0, bp - B), (0, 0)))
    xs = xf.reshape(ncores, grid_i, block_b, C * H * W)

    body = functools.partial(_flat_kernel, C=C, H=H, W=W,
                             oh1=oh1, oh2=oh2, oh3=oh3)

    ow1 = (W - _KSIZE) // 2 + 1
    ow2 = ow1 - (_KSIZE - 1)
    ow3 = ow2 - (_KSIZE - 1)
    flops = 2 * B * (oh1 * ow1 * 16 * (C * _KSIZE * _KSIZE)
                     + oh2 * ow2 * 32 * (16 * _KSIZE * _KSIZE)
                     + oh3 * ow3 * 64 * (32 * _KSIZE * _KSIZE)
                     + no * (64 * oh3 * ow3))
    bytes_accessed = (int(np.prod(xs.shape)) * 2 + bp * no * 4
                      + sum(int(a.size) * a.dtype.itemsize
                            for a in (t1, b1, t2, b2, t3, b3, wl, bl)))

    def full(a):
        nd = a.ndim
        return pl.BlockSpec(a.shape, lambda c, g, _nd=nd: (0,) * _nd)

    out = pl.pallas_call(
        body,
        out_shape=jax.ShapeDtypeStruct((ncores, grid_i, block_b, no),
                                       jnp.float32),
        grid=(ncores, grid_i),
        in_specs=[
            pl.BlockSpec((1, 1, block_b, C * H * W),
                         lambda c, g: (c, g, 0, 0)),
            full(t1p), full(b1), full(t2), full(b2), full(t3), full(b3),
            full(wl), full(bl),
        ],
        out_specs=pl.BlockSpec((1, 1, block_b, no),
                               lambda c, g: (c, g, 0, 0)),
        compiler_params=pltpu.CompilerParams(
            dimension_semantics=("arbitrary", "arbitrary")),
        cost_estimate=pl.CostEstimate(flops=flops, transcendentals=0,
                                      bytes_accessed=bytes_accessed),
    )(xs, t1p, b1, t2, b2, t3, b3, wl, bl)

    return out.reshape(bp, no)[:B, :no]


# speedup vs baseline: 1.0293x; 1.0182x over previous
import functools

import numpy as np
import jax
import jax.numpy as jnp
from jax.experimental import pallas as pl
from jax.experimental.pallas import tpu as pltpu

_KSIZE = 4
_SLOPE = 0.01


def _flat_kernel(x_ref, t1_ref, b1_ref, t2_ref, b2_ref, t3_ref, b3_ref,
                 wl_ref, bl_ref, o_ref, *, C, H, W, oh1, oh2, oh3):
    f32 = jnp.float32
    bf16 = jnp.bfloat16

    def lrelu(v):
        return jnp.where(v > 0, v, _SLOPE * v)

    n1 = t1_ref.shape[1]
    n2 = t2_ref.shape[1]
    n3 = t3_ref.shape[1]

    u = x_ref[0, 0]

    a1_parts = []
    for r in range(oh1):
        acc = b1_ref[...].astype(f32)
        for c in range(C):
            lo = c * H * W + 2 * r * W
            acc = acc + jnp.dot(u[:, lo:lo + _KSIZE * W],
                                t1_ref[c * _KSIZE * W:(c + 1) * _KSIZE * W, :],
                                preferred_element_type=f32)
        a1_parts.append(lrelu(acc).astype(bf16))
    a1 = jnp.concatenate(a1_parts, axis=1)

    a2_parts = [
        lrelu(jnp.dot(a1[:, r * n1:(r + _KSIZE) * n1], t2_ref[...],
                      preferred_element_type=f32) + b2_ref[...]).astype(bf16)
        for r in range(oh2)]
    a2 = jnp.concatenate(a2_parts, axis=1)

    a3_parts = [
        lrelu(jnp.dot(a2[:, r * n2:(r + _KSIZE) * n2], t3_ref[...],
                      preferred_element_type=f32) + b3_ref[...]).astype(bf16)
        for r in range(oh3)]
    a3 = jnp.concatenate(a3_parts, axis=1)

    y = jnp.dot(a3, wl_ref[...], preferred_element_type=f32) + bl_ref[...]
    o_ref[0, 0] = y


def kernel(x, t1, b1, t2, b2, t3, b3, wl, bl):
    B, C, H, W = x.shape
    oh1 = (H - _KSIZE) // 2 + 1
    oh2 = oh1 - (_KSIZE - 1)
    oh3 = oh2 - (_KSIZE - 1)
    wcp = t1.shape[0] // _KSIZE
    no = bl.shape[1]

    ncores = 1
    block_b = max(1, min(512, -(-B // 2)))
    block_b = min(block_b, B)
    grid_b = -(-B // block_b)
    bp = grid_b * block_b
    grid_i = grid_b

    n1 = t1.shape[1]
    t1p = (t1.reshape(_KSIZE, wcp, n1)[:, :C * W, :]
           .reshape(_KSIZE, W, C, n1)
           .transpose(2, 0, 1, 3)
           .reshape(C * _KSIZE * W, n1))

    xf = x.reshape(B, C * H * W).astype(jnp.bfloat16)
    if bp > B:
        xf = jnp.pad(xf, ((0, bp - B), (0, 0)))
    xs = xf.reshape(ncores, grid_i, block_b, C * H * W)

    body = functools.partial(_flat_kernel, C=C, H=H, W=W,
                             oh1=oh1, oh2=oh2, oh3=oh3)

    ow1 = (W - _KSIZE) // 2 + 1
    ow2 = ow1 - (_KSIZE - 1)
    ow3 = ow2 - (_KSIZE - 1)
    flops = 2 * B * (oh1 * ow1 * 16 * (C * _KSIZE * _KSIZE)
                     + oh2 * ow2 * 32 * (16 * _KSIZE * _KSIZE)
                     + oh3 * ow3 * 64 * (32 * _KSIZE * _KSIZE)
                     + no * (64 * oh3 * ow3))
    bytes_accessed = (int(np.prod(xs.shape)) * 2 + bp * no * 4
                      + sum(int(a.size) * a.dtype.itemsize
                            for a in (t1, b1, t2, b2, t3, b3, wl, bl)))

    def full(a):
        nd = a.ndim
        return pl.BlockSpec(a.shape, lambda c, g, _nd=nd: (0,) * _nd)

    out = pl.pallas_call(
        body,
        out_shape=jax.ShapeDtypeStruct((ncores, grid_i, block_b, no),
                                       jnp.float32),
        grid=(ncores, grid_i),
        in_specs=[
            pl.BlockSpec((1, 1, block_b, C * H * W),
                         lambda c, g: (c, g, 0, 0)),
            full(t1p), full(b1), full(t2), full(b2), full(t3), full(b3),
            full(wl), full(bl),
        ],
        out_specs=pl.BlockSpec((1, 1, block_b, no),
                               lambda c, g: (c, g, 0, 0)),
        compiler_params=pltpu.CompilerParams(
            dimension_semantics=("arbitrary", "arbitrary")),
        cost_estimate=pl.CostEstimate(flops=flops, transcendentals=0,
                                      bytes_accessed=bytes_accessed),
    )(xs, t1p, b1, t2, b2, t3, b3, wl, bl)

    return out.reshape(bp, no)[:B, :no]
